# baseline (speedup 1.0000x reference)
import functools

import jax
import jax.numpy as jnp
from jax import lax
from jax.experimental import pallas as pl
from jax.experimental.pallas import tpu as pltpu
from jax.experimental.pallas import tpu_sc as plsc

N_NODES = 10000
NP = 10240
NPA = NP + 16
N_EDGES = 160000
EP = 163840
D = 256
H = 128
EPS = 1e-5
K = 128
NSC = 2
NT = 16
RPT = NP // NT
RPTA = NPA // NT
BN = 512
NBI = NP // BN

ET_E = EP // NT
NB_E = ET_E // K
CH = 16
ET_D = EP // (NSC * NT)
NB_D = ET_D // K

_MESH = plsc.VectorSubcoreMesh(
    core_axis_name="c", subcore_axis_name="s", num_cores=NSC, num_subcores=NT
)


@functools.partial(
    pl.kernel,
    out_type=jax.ShapeDtypeStruct((NSC, NP, H), jnp.float32),
    mesh=_MESH,
    scratch_types=[
        pltpu.VMEM((NB_D, K), jnp.int32),
        pltpu.VMEM((K, H), jnp.float32),
        pltpu.VMEM_SHARED((NPA, H), jnp.float32),
    ],
)
def _sc_degree(col_hbm, zeros_hbm, ones_hbm, out_hbm, idx_v, ones_v, acc):
    cid = lax.axis_index("c")
    sid = lax.axis_index("s")
    pltpu.sync_copy(zeros_hbm.at[pl.ds(sid * RPT, RPT)],
                    acc.at[pl.ds(sid * RPT, RPT)])
    pltpu.sync_copy(ones_hbm, ones_v)
    tile = cid * NT + sid
    pltpu.sync_copy(col_hbm.at[tile], idx_v)
    plsc.subcore_barrier()

    def body(b, carry):
        pltpu.sync_copy(ones_v, acc.at[idx_v.at[b]], add=True)
        return carry

    lax.fori_loop(0, NB_D, body, 0)
    plsc.subcore_barrier()
    pltpu.sync_copy(acc.at[pl.ds(sid * RPT, RPT)],
                    out_hbm.at[cid, pl.ds(sid * RPT, RPT)])


@functools.partial(
    pl.kernel,
    out_type=jax.ShapeDtypeStruct((NSC, NP, H), jnp.float32),
    mesh=_MESH,
    scratch_types=[
        pltpu.VMEM((CH, K), jnp.int32),
        pltpu.VMEM((CH, K), jnp.int32),
        pltpu.VMEM((K, H), jnp.float32),
        pltpu.VMEM((K, H), jnp.float32),
        pltpu.VMEM_SHARED((NPA, H), jnp.float32),
        pltpu.SemaphoreType.DMA,
        pltpu.SemaphoreType.DMA,
        pltpu.SemaphoreType.DMA,
        pltpu.SemaphoreType.DMA,
    ],
)
def _sc_edge_sum(row_hbm, col_hbm, y_hbm, out_hbm, idxr, idxc, gbuf0, gbuf1,
                 acc, semg0, semg1, sems0, sems1):
    cid = lax.axis_index("c")
    sid = lax.axis_index("s")
    pltpu.sync_copy(y_hbm.at[pl.ds(cid * NP + sid * RPT, RPT)],
                    acc.at[pl.ds(sid * RPT, RPT)])
    plsc.subcore_barrier()

    def chunk(ch, carry):
        pltpu.sync_copy(row_hbm.at[cid, sid, pl.ds(ch * CH, CH)], idxr)
        pltpu.sync_copy(col_hbm.at[sid, pl.ds(ch * CH, CH)], idxc)
        pltpu.async_copy(y_hbm.at[idxr.at[0]], gbuf0, semg0)
        pltpu.async_copy(y_hbm.at[idxr.at[1]], gbuf1, semg1)

        def body(i, c2):
            b0 = 2 * i
            pltpu.make_async_copy(y_hbm.at[idxr.at[b0]], gbuf0, semg0).wait()
            pltpu.async_copy(gbuf0, acc.at[idxc.at[b0]], sems0, add=True)
            pltpu.make_async_copy(y_hbm.at[idxr.at[b0 + 1]], gbuf1, semg1).wait()
            pltpu.async_copy(gbuf1, acc.at[idxc.at[b0 + 1]], sems1, add=True)

            @pl.when(b0 + 2 < CH)
            def _():
                pltpu.make_async_copy(gbuf0, acc.at[idxc.at[b0]], sems0).wait()
                pltpu.async_copy(y_hbm.at[idxr.at[b0 + 2]], gbuf0, semg0)

            @pl.when(b0 + 3 < CH)
            def _():
                pltpu.make_async_copy(gbuf1, acc.at[idxc.at[b0 + 1]], sems1).wait()
                pltpu.async_copy(y_hbm.at[idxr.at[b0 + 3]], gbuf1, semg1)

            return c2

        lax.fori_loop(0, CH // 2, body, 0)
        pltpu.make_async_copy(gbuf0, acc.at[idxc.at[CH - 2]], sems0).wait()
        pltpu.make_async_copy(gbuf1, acc.at[idxc.at[CH - 1]], sems1).wait()
        return carry

    lax.fori_loop(0, NB_E // CH, chunk, 0)
    plsc.subcore_barrier()
    pltpu.sync_copy(acc.at[pl.ds(sid * RPT, RPT)],
                    out_hbm.at[cid, pl.ds(sid * RPT, RPT)])


def _tc_first_body(x_ref, w_ref, dp_ref, y_ref, s_ref):
    deg = jnp.sum(dp_ref[...], axis=(0, 2)) * (1.0 / H) + 1.0
    s = (1.0 / jnp.sqrt(deg))[:, None]
    y = jnp.dot(x_ref[...], w_ref[...], preferred_element_type=jnp.float32) * s
    y_ref[0] = y[:, :H]
    y_ref[1] = y[:, H:]
    s_ref[...] = s


def _tc_mid_body(z_ref, s_ref, b_ref, g_ref, be_ref, w_ref, y_ref):
    s = s_ref[...]
    u = jnp.concatenate([z_ref[0], z_ref[1]], axis=1) * s + b_ref[...]
    mu = jnp.mean(u, axis=1, keepdims=True)
    var = jnp.mean((u - mu) ** 2, axis=1, keepdims=True)
    t = g_ref[...] * (u - mu) / jnp.sqrt(var + EPS) + be_ref[...]
    t = jnp.maximum(t, 0.0)
    y = jnp.dot(t, w_ref[...], preferred_element_type=jnp.float32) * s
    y_ref[0] = y[:, :H]
    y_ref[1] = y[:, H:]


def _tc_final_body(z_ref, s_ref, b_ref, g_ref, be_ref, o_ref):
    s = s_ref[...]
    u = jnp.concatenate([z_ref[0], z_ref[1]], axis=1) * s + b_ref[...]
    mu = jnp.mean(u, axis=1, keepdims=True)
    var = jnp.mean((u - mu) ** 2, axis=1, keepdims=True)
    t = g_ref[...] * (u - mu) / jnp.sqrt(var + EPS) + be_ref[...]
    o_ref[...] = jnp.maximum(t, 0.0)


_VEC_SPEC = pl.BlockSpec((1, D), lambda i: (0, 0))
_Z_SPEC = pl.BlockSpec((NSC, BN, H), lambda i: (0, i, 0))
_S_SPEC = pl.BlockSpec((BN, 1), lambda i: (i, 0))
_W_SPEC = pl.BlockSpec((D, D), lambda i: (0, 0))

_tc_first = pl.pallas_call(
    _tc_first_body,
    grid=(NBI,),
    in_specs=[
        pl.BlockSpec((BN, D), lambda i: (i, 0)),
        _W_SPEC,
        pl.BlockSpec((NSC, BN, H), lambda i: (0, i, 0)),
    ],
    out_specs=[_Z_SPEC, _S_SPEC],
    out_shape=[
        jax.ShapeDtypeStruct((NSC, NP, H), jnp.float32),
        jax.ShapeDtypeStruct((NP, 1), jnp.float32),
    ],
)

_tc_mid = pl.pallas_call(
    _tc_mid_body,
    grid=(NBI,),
    in_specs=[_Z_SPEC, _S_SPEC, _VEC_SPEC, _VEC_SPEC, _VEC_SPEC, _W_SPEC],
    out_specs=_Z_SPEC,
    out_shape=jax.ShapeDtypeStruct((NSC, NP, H), jnp.float32),
)

_tc_final = pl.pallas_call(
    _tc_final_body,
    grid=(NBI,),
    in_specs=[_Z_SPEC, _S_SPEC, _VEC_SPEC, _VEC_SPEC, _VEC_SPEC],
    out_specs=pl.BlockSpec((BN, D), lambda i: (i, 0)),
    out_shape=jax.ShapeDtypeStruct((NP, D), jnp.float32),
)


def kernel(x, edge_index, W1, b1, g1, be1, W2, b2, g2, be2,
           W3, b3, g3, be3, W4, b4, g4, be4):
    f32 = jnp.float32
    ei = edge_index.astype(jnp.int32)
    pad_e = EP - N_EDGES
    rowp = jnp.concatenate([ei[0], jnp.zeros((pad_e,), jnp.int32)])
    colp = jnp.concatenate([ei[1], jnp.full((pad_e,), NP, jnp.int32)])
    rowboth = jnp.stack([rowp, rowp + NP]).reshape(NSC, NT, NB_E, K)
    col_e = colp.reshape(NT, NB_E, K)
    col_d = colp.reshape(NSC * NT, NB_D, K)
    xp = jnp.pad(x, ((0, NP - N_NODES), (0, 0)))

    dparts = _sc_degree(col_d, jnp.zeros((NP, H), f32), jnp.ones((K, H), f32))
    y, s = _tc_first(xp, W1, dparts)

    params = [(b1, g1, be1, W2), (b2, g2, be2, W3), (b3, g3, be3, W4)]
    for (b, g, be, w_next) in params:
        z = _sc_edge_sum(rowboth, col_e, y.reshape(NSC * NP, H))
        y = _tc_mid(z, s, b.reshape(1, D), g.reshape(1, D), be.reshape(1, D),
                    w_next)
    z = _sc_edge_sum(rowboth, col_e, y.reshape(NSC * NP, H))
    h = _tc_final(z, s, b4.reshape(1, D), g4.reshape(1, D), be4.reshape(1, D))
    return h[:N_NODES]


# speedup vs baseline: 1.0084x; 1.0084x over previous
import functools

import jax
import jax.numpy as jnp
from jax import lax
from jax.experimental import pallas as pl
from jax.experimental.pallas import tpu as pltpu
from jax.experimental.pallas import tpu_sc as plsc

N_NODES = 10000
NP = 10240
NPA = NP + 16
N_EDGES = 160000
EP = 163840
D = 256
H = 128
EPS = 1e-5
K = 80
NSLOT = 4
NSC = 2
NT = 16
RPT = NP // NT
RPTA = NPA // NT
BN = 512
NBI = NP // BN

ET_E = EP // NT
NB_E = ET_E // K
CH = 16
ET_D = EP // (NSC * NT)
NB_D = ET_D // K

_MESH = plsc.VectorSubcoreMesh(
    core_axis_name="c", subcore_axis_name="s", num_cores=NSC, num_subcores=NT
)


@functools.partial(
    pl.kernel,
    out_type=jax.ShapeDtypeStruct((NSC, NP, H), jnp.float32),
    mesh=_MESH,
    scratch_types=[
        pltpu.VMEM((NB_D, K), jnp.int32),
        pltpu.VMEM((K, H), jnp.float32),
        pltpu.VMEM_SHARED((NPA, H), jnp.float32),
    ],
)
def _sc_degree(col_hbm, zeros_hbm, ones_hbm, out_hbm, idx_v, ones_v, acc):
    cid = lax.axis_index("c")
    sid = lax.axis_index("s")
    pltpu.sync_copy(zeros_hbm.at[pl.ds(sid * RPT, RPT)],
                    acc.at[pl.ds(sid * RPT, RPT)])
    pltpu.sync_copy(ones_hbm, ones_v)
    tile = cid * NT + sid
    pltpu.sync_copy(col_hbm.at[tile], idx_v)
    plsc.subcore_barrier()

    def body(b, carry):
        pltpu.sync_copy(ones_v, acc.at[idx_v.at[b]], add=True)
        return carry

    lax.fori_loop(0, NB_D, body, 0)
    plsc.subcore_barrier()
    pltpu.sync_copy(acc.at[pl.ds(sid * RPT, RPT)],
                    out_hbm.at[cid, pl.ds(sid * RPT, RPT)])


@functools.partial(
    pl.kernel,
    out_type=jax.ShapeDtypeStruct((NSC, NP, H), jnp.float32),
    mesh=_MESH,
    scratch_types=[
        pltpu.VMEM((CH, K), jnp.int32),
        pltpu.VMEM((CH, K), jnp.int32),
        [pltpu.VMEM((K, H), jnp.float32)] * NSLOT,
        pltpu.VMEM_SHARED((NPA, H), jnp.float32),
        [pltpu.SemaphoreType.DMA] * NSLOT,
        [pltpu.SemaphoreType.DMA] * NSLOT,
    ],
)
def _sc_edge_sum(row_hbm, col_hbm, y_hbm, out_hbm, idxr, idxc, gbufs,
                 acc, semgs, semss):
    cid = lax.axis_index("c")
    sid = lax.axis_index("s")
    pltpu.sync_copy(y_hbm.at[pl.ds(cid * NP + sid * RPT, RPT)],
                    acc.at[pl.ds(sid * RPT, RPT)])
    plsc.subcore_barrier()

    def chunk(ch, carry):
        pltpu.sync_copy(row_hbm.at[cid, sid, pl.ds(ch * CH, CH)], idxr)
        pltpu.sync_copy(col_hbm.at[sid, pl.ds(ch * CH, CH)], idxc)
        for j in range(NSLOT):
            pltpu.async_copy(y_hbm.at[idxr.at[j]], gbufs[j], semgs[j])

        def body(i, c2):
            b = NSLOT * i
            for j in range(NSLOT):
                pltpu.make_async_copy(y_hbm.at[idxr.at[b + j]], gbufs[j],
                                      semgs[j]).wait()
                pltpu.async_copy(gbufs[j], acc.at[idxc.at[b + j]], semss[j],
                                 add=True)
            for j in range(NSLOT):
                @pl.when(b + NSLOT + j < CH)
                def _(j=j):
                    pltpu.make_async_copy(gbufs[j], acc.at[idxc.at[b + j]],
                                          semss[j]).wait()
                    pltpu.async_copy(y_hbm.at[idxr.at[b + NSLOT + j]],
                                     gbufs[j], semgs[j])

            return c2

        lax.fori_loop(0, CH // NSLOT, body, 0)
        for j in range(NSLOT):
            pltpu.make_async_copy(gbufs[j], acc.at[idxc.at[CH - NSLOT + j]],
                                  semss[j]).wait()
        return carry

    lax.fori_loop(0, NB_E // CH, chunk, 0)
    plsc.subcore_barrier()
    pltpu.sync_copy(acc.at[pl.ds(sid * RPT, RPT)],
                    out_hbm.at[cid, pl.ds(sid * RPT, RPT)])


def _tc_first_body(x_ref, w_ref, dp_ref, y_ref, s_ref):
    deg = jnp.sum(dp_ref[...], axis=(0, 2)) * (1.0 / H) + 1.0
    s = (1.0 / jnp.sqrt(deg))[:, None]
    y = jnp.dot(x_ref[...], w_ref[...], preferred_element_type=jnp.float32) * s
    y_ref[0] = y[:, :H]
    y_ref[1] = y[:, H:]
    s_ref[...] = s


def _tc_mid_body(z_ref, s_ref, b_ref, g_ref, be_ref, w_ref, y_ref):
    s = s_ref[...]
    u = jnp.concatenate([z_ref[0], z_ref[1]], axis=1) * s + b_ref[...]
    mu = jnp.mean(u, axis=1, keepdims=True)
    var = jnp.mean((u - mu) ** 2, axis=1, keepdims=True)
    t = g_ref[...] * (u - mu) / jnp.sqrt(var + EPS) + be_ref[...]
    t = jnp.maximum(t, 0.0)
    y = jnp.dot(t, w_ref[...], preferred_element_type=jnp.float32) * s
    y_ref[0] = y[:, :H]
    y_ref[1] = y[:, H:]


def _tc_final_body(z_ref, s_ref, b_ref, g_ref, be_ref, o_ref):
    s = s_ref[...]
    u = jnp.concatenate([z_ref[0], z_ref[1]], axis=1) * s + b_ref[...]
    mu = jnp.mean(u, axis=1, keepdims=True)
    var = jnp.mean((u - mu) ** 2, axis=1, keepdims=True)
    t = g_ref[...] * (u - mu) / jnp.sqrt(var + EPS) + be_ref[...]
    o_ref[...] = jnp.maximum(t, 0.0)


_VEC_SPEC = pl.BlockSpec((1, D), lambda i: (0, 0))
_Z_SPEC = pl.BlockSpec((NSC, BN, H), lambda i: (0, i, 0))
_S_SPEC = pl.BlockSpec((BN, 1), lambda i: (i, 0))
_W_SPEC = pl.BlockSpec((D, D), lambda i: (0, 0))

_tc_first = pl.pallas_call(
    _tc_first_body,
    grid=(NBI,),
    in_specs=[
        pl.BlockSpec((BN, D), lambda i: (i, 0)),
        _W_SPEC,
        pl.BlockSpec((NSC, BN, H), lambda i: (0, i, 0)),
    ],
    out_specs=[_Z_SPEC, _S_SPEC],
    out_shape=[
        jax.ShapeDtypeStruct((NSC, NP, H), jnp.float32),
        jax.ShapeDtypeStruct((NP, 1), jnp.float32),
    ],
)

_tc_mid = pl.pallas_call(
    _tc_mid_body,
    grid=(NBI,),
    in_specs=[_Z_SPEC, _S_SPEC, _VEC_SPEC, _VEC_SPEC, _VEC_SPEC, _W_SPEC],
    out_specs=_Z_SPEC,
    out_shape=jax.ShapeDtypeStruct((NSC, NP, H), jnp.float32),
)

_tc_final = pl.pallas_call(
    _tc_final_body,
    grid=(NBI,),
    in_specs=[_Z_SPEC, _S_SPEC, _VEC_SPEC, _VEC_SPEC, _VEC_SPEC],
    out_specs=pl.BlockSpec((BN, D), lambda i: (i, 0)),
    out_shape=jax.ShapeDtypeStruct((NP, D), jnp.float32),
)


def kernel(x, edge_index, W1, b1, g1, be1, W2, b2, g2, be2,
           W3, b3, g3, be3, W4, b4, g4, be4):
    f32 = jnp.float32
    ei = edge_index.astype(jnp.int32)
    pad_e = EP - N_EDGES
    rowp = jnp.concatenate([ei[0], jnp.zeros((pad_e,), jnp.int32)])
    colp = jnp.concatenate([ei[1], jnp.full((pad_e,), NP, jnp.int32)])
    rowboth = jnp.stack([rowp, rowp + NP]).reshape(NSC, NT, NB_E, K)
    col_e = colp.reshape(NT, NB_E, K)
    col_d = colp.reshape(NSC * NT, NB_D, K)
    xp = jnp.pad(x, ((0, NP - N_NODES), (0, 0)))

    dparts = _sc_degree(col_d, jnp.zeros((NP, H), f32), jnp.ones((K, H), f32))
    y, s = _tc_first(xp, W1, dparts)

    params = [(b1, g1, be1, W2), (b2, g2, be2, W3), (b3, g3, be3, W4)]
    for (b, g, be, w_next) in params:
        z = _sc_edge_sum(rowboth, col_e, y.reshape(NSC * NP, H))
        y = _tc_mid(z, s, b.reshape(1, D), g.reshape(1, D), be.reshape(1, D),
                    w_next)
    z = _sc_edge_sum(rowboth, col_e, y.reshape(NSC * NP, H))
    h = _tc_final(z, s, b4.reshape(1, D), g4.reshape(1, D), be4.reshape(1, D))
    return h[:N_NODES]


# speedup vs baseline: 1.0881x; 1.0791x over previous
import functools

import jax
import jax.numpy as jnp
from jax import lax
from jax.experimental import pallas as pl
from jax.experimental.pallas import tpu as pltpu
from jax.experimental.pallas import tpu_sc as plsc

N_NODES = 10000
NP = 10240
NPA = NP + 16
N_EDGES = 160000
EP = 163840
D = 256
H = 128
EPS = 1e-5
K = 128
NSC = 2
NT = 16
RPT = NP // NT
RPTA = NPA // NT
BN = 512
NBI = NP // BN

ET_E = EP // NT
NB_E = ET_E // K
CH = 40
ET_D = EP // (NSC * NT)
NB_D = ET_D // K

_MESH = plsc.VectorSubcoreMesh(
    core_axis_name="c", subcore_axis_name="s", num_cores=NSC, num_subcores=NT
)


@functools.partial(
    pl.kernel,
    out_type=jax.ShapeDtypeStruct((NSC, NP, H), jnp.float32),
    mesh=_MESH,
    scratch_types=[
        pltpu.VMEM((NB_D, K), jnp.int32),
        pltpu.VMEM((K, H), jnp.float32),
        pltpu.VMEM_SHARED((NPA, H), jnp.float32),
    ],
)
def _sc_degree(col_hbm, zeros_hbm, ones_hbm, out_hbm, idx_v, ones_v, acc):
    cid = lax.axis_index("c")
    sid = lax.axis_index("s")
    pltpu.sync_copy(zeros_hbm.at[pl.ds(sid * RPT, RPT)],
                    acc.at[pl.ds(sid * RPT, RPT)])
    pltpu.sync_copy(ones_hbm, ones_v)
    tile = cid * NT + sid
    pltpu.sync_copy(col_hbm.at[tile], idx_v)
    plsc.subcore_barrier()

    def body(b, carry):
        pltpu.sync_copy(ones_v, acc.at[idx_v.at[b]], add=True)
        return carry

    lax.fori_loop(0, NB_D, body, 0)
    plsc.subcore_barrier()
    pltpu.sync_copy(acc.at[pl.ds(sid * RPT, RPT)],
                    out_hbm.at[cid, pl.ds(sid * RPT, RPT)])


@functools.partial(
    pl.kernel,
    out_type=jax.ShapeDtypeStruct((NSC, NP, H), jnp.float32),
    mesh=_MESH,
    scratch_types=[
        pltpu.VMEM((CH, K), jnp.int32),
        pltpu.VMEM((CH, K), jnp.int32),
        pltpu.VMEM((K, H), jnp.float32),
        pltpu.VMEM((K, H), jnp.float32),
        pltpu.VMEM_SHARED((NPA, H), jnp.float32),
        pltpu.SemaphoreType.DMA,
        pltpu.SemaphoreType.DMA,
    ],
)
def _sc_edge_sum(row_hbm, col_hbm, y_hbm, out_hbm, idxr, idxc, gbuf0, gbuf1,
                 acc, sem0, sem1):
    cid = lax.axis_index("c")
    sid = lax.axis_index("s")
    pltpu.sync_copy(y_hbm.at[pl.ds(cid * NP + sid * RPT, RPT)],
                    acc.at[pl.ds(sid * RPT, RPT)])
    plsc.subcore_barrier()

    def chunk(ch, carry):
        pltpu.sync_copy(row_hbm.at[cid, sid, pl.ds(ch * CH, CH)], idxr)
        pltpu.sync_copy(col_hbm.at[sid, pl.ds(ch * CH, CH)], idxc)
        pltpu.async_copy(y_hbm.at[idxr.at[0]], gbuf0, sem0)

        def body(i, c2):
            b0 = 2 * i
            pltpu.async_copy(y_hbm.at[idxr.at[b0 + 1]], gbuf1, sem1)
            pltpu.make_async_copy(y_hbm.at[idxr.at[b0]], gbuf0, sem0).wait()
            pltpu.sync_copy(gbuf0, acc.at[idxc.at[b0]], add=True)

            @pl.when(b0 + 2 < CH)
            def _():
                pltpu.async_copy(y_hbm.at[idxr.at[b0 + 2]], gbuf0, sem0)

            pltpu.make_async_copy(y_hbm.at[idxr.at[b0 + 1]], gbuf1, sem1).wait()
            pltpu.sync_copy(gbuf1, acc.at[idxc.at[b0 + 1]], add=True)
            return c2

        lax.fori_loop(0, CH // 2, body, 0)
        return carry

    lax.fori_loop(0, NB_E // CH, chunk, 0)
    plsc.subcore_barrier()
    pltpu.sync_copy(acc.at[pl.ds(sid * RPT, RPT)],
                    out_hbm.at[cid, pl.ds(sid * RPT, RPT)])


def _tc_first_body(x_ref, w_ref, dp_ref, y_ref, s_ref):
    deg = jnp.sum(dp_ref[...], axis=(0, 2)) * (1.0 / H) + 1.0
    s = (1.0 / jnp.sqrt(deg))[:, None]
    y = jnp.dot(x_ref[...], w_ref[...], preferred_element_type=jnp.float32) * s
    y_ref[0] = y[:, :H]
    y_ref[1] = y[:, H:]
    s_ref[...] = s


def _tc_mid_body(z_ref, s_ref, b_ref, g_ref, be_ref, w_ref, y_ref):
    s = s_ref[...]
    u = jnp.concatenate([z_ref[0], z_ref[1]], axis=1) * s + b_ref[...]
    mu = jnp.mean(u, axis=1, keepdims=True)
    var = jnp.mean((u - mu) ** 2, axis=1, keepdims=True)
    t = g_ref[...] * (u - mu) / jnp.sqrt(var + EPS) + be_ref[...]
    t = jnp.maximum(t, 0.0)
    y = jnp.dot(t, w_ref[...], preferred_element_type=jnp.float32) * s
    y_ref[0] = y[:, :H]
    y_ref[1] = y[:, H:]


def _tc_final_body(z_ref, s_ref, b_ref, g_ref, be_ref, o_ref):
    s = s_ref[...]
    u = jnp.concatenate([z_ref[0], z_ref[1]], axis=1) * s + b_ref[...]
    mu = jnp.mean(u, axis=1, keepdims=True)
    var = jnp.mean((u - mu) ** 2, axis=1, keepdims=True)
    t = g_ref[...] * (u - mu) / jnp.sqrt(var + EPS) + be_ref[...]
    o_ref[...] = jnp.maximum(t, 0.0)


_VEC_SPEC = pl.BlockSpec((1, D), lambda i: (0, 0))
_Z_SPEC = pl.BlockSpec((NSC, BN, H), lambda i: (0, i, 0))
_S_SPEC = pl.BlockSpec((BN, 1), lambda i: (i, 0))
_W_SPEC = pl.BlockSpec((D, D), lambda i: (0, 0))

_tc_first = pl.pallas_call(
    _tc_first_body,
    grid=(NBI,),
    in_specs=[
        pl.BlockSpec((BN, D), lambda i: (i, 0)),
        _W_SPEC,
        pl.BlockSpec((NSC, BN, H), lambda i: (0, i, 0)),
    ],
    out_specs=[_Z_SPEC, _S_SPEC],
    out_shape=[
        jax.ShapeDtypeStruct((NSC, NP, H), jnp.float32),
        jax.ShapeDtypeStruct((NP, 1), jnp.float32),
    ],
)

_tc_mid = pl.pallas_call(
    _tc_mid_body,
    grid=(NBI,),
    in_specs=[_Z_SPEC, _S_SPEC, _VEC_SPEC, _VEC_SPEC, _VEC_SPEC, _W_SPEC],
    out_specs=_Z_SPEC,
    out_shape=jax.ShapeDtypeStruct((NSC, NP, H), jnp.float32),
)

_tc_final = pl.pallas_call(
    _tc_final_body,
    grid=(NBI,),
    in_specs=[_Z_SPEC, _S_SPEC, _VEC_SPEC, _VEC_SPEC, _VEC_SPEC],
    out_specs=pl.BlockSpec((BN, D), lambda i: (i, 0)),
    out_shape=jax.ShapeDtypeStruct((NP, D), jnp.float32),
)


def kernel(x, edge_index, W1, b1, g1, be1, W2, b2, g2, be2,
           W3, b3, g3, be3, W4, b4, g4, be4):
    f32 = jnp.float32
    ei = edge_index.astype(jnp.int32)
    pad_e = EP - N_EDGES
    rowp = jnp.concatenate([ei[0], jnp.zeros((pad_e,), jnp.int32)])
    colp = jnp.concatenate([ei[1], jnp.full((pad_e,), NP, jnp.int32)])
    rowboth = jnp.stack([rowp, rowp + NP]).reshape(NSC, NT, NB_E, K)
    col_e = colp.reshape(NT, NB_E, K)
    col_d = colp.reshape(NSC * NT, NB_D, K)
    xp = jnp.pad(x, ((0, NP - N_NODES), (0, 0)))

    dparts = _sc_degree(col_d, jnp.zeros((NP, H), f32), jnp.ones((K, H), f32))
    y, s = _tc_first(xp, W1, dparts)

    params = [(b1, g1, be1, W2), (b2, g2, be2, W3), (b3, g3, be3, W4)]
    for (b, g, be, w_next) in params:
        z = _sc_edge_sum(rowboth, col_e, y.reshape(NSC * NP, H))
        y = _tc_mid(z, s, b.reshape(1, D), g.reshape(1, D), be.reshape(1, D),
                    w_next)
    z = _sc_edge_sum(rowboth, col_e, y.reshape(NSC * NP, H))
    h = _tc_final(z, s, b4.reshape(1, D), g4.reshape(1, D), be4.reshape(1, D))
    return h[:N_NODES]


# speedup vs baseline: 1.1898x; 1.0935x over previous
import functools

import jax
import jax.numpy as jnp
from jax import lax
from jax.experimental import pallas as pl
from jax.experimental.pallas import tpu as pltpu
from jax.experimental.pallas import tpu_sc as plsc

N_NODES = 10000
NP = 10240
NPA = NP + 16
N_EDGES = 160000
EP = 163840
D = 256
H = 128
EPS = 1e-5
K = 128
NSC = 2
NT = 16
RPT = NP // NT
RPTA = NPA // NT
BN = 512
NBI = NP // BN

ET_E = EP // NT
NB_E = ET_E // K
CH = 40
ET_D = EP // (NSC * NT)
NB_D = ET_D // K

_MESH = plsc.VectorSubcoreMesh(
    core_axis_name="c", subcore_axis_name="s", num_cores=NSC, num_subcores=NT
)


@functools.partial(
    pl.kernel,
    out_type=jax.ShapeDtypeStruct((NSC, NP, H), jnp.float32),
    mesh=_MESH,
    scratch_types=[
        pltpu.VMEM((NB_D, K), jnp.int32),
        pltpu.VMEM((K, H), jnp.float32),
        pltpu.VMEM_SHARED((NPA, H), jnp.float32),
    ],
)
def _sc_degree(col_hbm, zeros_hbm, ones_hbm, out_hbm, idx_v, ones_v, acc):
    cid = lax.axis_index("c")
    sid = lax.axis_index("s")
    pltpu.sync_copy(zeros_hbm.at[pl.ds(sid * RPT, RPT)],
                    acc.at[pl.ds(sid * RPT, RPT)])
    pltpu.sync_copy(ones_hbm, ones_v)
    tile = cid * NT + sid
    pltpu.sync_copy(col_hbm.at[tile], idx_v)
    plsc.subcore_barrier()

    def body(b, carry):
        pltpu.sync_copy(ones_v, acc.at[idx_v.at[b]], add=True)
        return carry

    lax.fori_loop(0, NB_D, body, 0)
    plsc.subcore_barrier()
    pltpu.sync_copy(acc.at[pl.ds(sid * RPT, RPT)],
                    out_hbm.at[cid, pl.ds(sid * RPT, RPT)])


@functools.partial(
    pl.kernel,
    out_type=jax.ShapeDtypeStruct((NSC, NP, H), jnp.float32),
    mesh=_MESH,
    scratch_types=[
        pltpu.VMEM((CH, K), jnp.int32),
        pltpu.VMEM((CH, K), jnp.int32),
        pltpu.VMEM((K, H), jnp.float32),
        pltpu.VMEM((K, H), jnp.float32),
        pltpu.VMEM_SHARED((NPA, H), jnp.float32),
        pltpu.SemaphoreType.DMA,
        pltpu.SemaphoreType.DMA,
    ],
)
def _sc_edge_sum(row_hbm, col_hbm, y_hbm, out_hbm, idxr, idxc, gbuf0, gbuf1,
                 acc, sem0, sem1):
    cid = lax.axis_index("c")
    sid = lax.axis_index("s")
    pltpu.sync_copy(y_hbm.at[pl.ds(cid * NP + sid * RPT, RPT)],
                    acc.at[pl.ds(sid * RPT, RPT)])
    plsc.subcore_barrier()

    def chunk(ch, carry):
        pltpu.sync_copy(row_hbm.at[cid, sid, pl.ds(ch * CH, CH)], idxr)
        pltpu.sync_copy(col_hbm.at[sid, pl.ds(ch * CH, CH)], idxc)
        pltpu.async_copy(y_hbm.at[idxr.at[0]], gbuf0, sem0)

        def body(i, c2):
            b0 = 2 * i
            pltpu.async_copy(y_hbm.at[idxr.at[b0 + 1]], gbuf1, sem1)
            pltpu.make_async_copy(y_hbm.at[idxr.at[b0]], gbuf0, sem0).wait()
            pltpu.sync_copy(gbuf0, acc.at[idxc.at[b0]], add=True)

            @pl.when(b0 + 2 < CH)
            def _():
                pltpu.async_copy(y_hbm.at[idxr.at[b0 + 2]], gbuf0, sem0)

            pltpu.make_async_copy(y_hbm.at[idxr.at[b0 + 1]], gbuf1, sem1).wait()
            pltpu.sync_copy(gbuf1, acc.at[idxc.at[b0 + 1]], add=True)
            return c2

        lax.fori_loop(0, CH // 2, body, 0)
        return carry

    lax.fori_loop(0, NB_E // CH, chunk, 0)
    plsc.subcore_barrier()
    pltpu.sync_copy(acc.at[pl.ds(sid * RPT, RPT)],
                    out_hbm.at[cid, pl.ds(sid * RPT, RPT)])


def _tc_mm_body(x_ref, w_ref, o_ref):
    o_ref[...] = jnp.dot(x_ref[...], w_ref[...],
                         preferred_element_type=jnp.float32)


def _tc_first_body(t_ref, dp_ref, y_ref, s_ref):
    deg = jnp.sum(dp_ref[...], axis=(0, 2)) * (1.0 / H) + 1.0
    s = (1.0 / jnp.sqrt(deg))[:, None]
    y = t_ref[...] * s
    y_ref[0] = y[:, :H]
    y_ref[1] = y[:, H:]
    s_ref[...] = s


def _tc_mid_body(z_ref, s_ref, b_ref, g_ref, be_ref, w_ref, y_ref):
    s = s_ref[...]
    u = jnp.concatenate([z_ref[0], z_ref[1]], axis=1) * s + b_ref[...]
    mu = jnp.mean(u, axis=1, keepdims=True)
    var = jnp.mean((u - mu) ** 2, axis=1, keepdims=True)
    t = g_ref[...] * (u - mu) / jnp.sqrt(var + EPS) + be_ref[...]
    t = jnp.maximum(t, 0.0)
    y = jnp.dot(t, w_ref[...], preferred_element_type=jnp.float32) * s
    y_ref[0] = y[:, :H]
    y_ref[1] = y[:, H:]


def _tc_final_body(z_ref, s_ref, b_ref, g_ref, be_ref, o_ref):
    s = s_ref[...]
    u = jnp.concatenate([z_ref[0], z_ref[1]], axis=1) * s + b_ref[...]
    mu = jnp.mean(u, axis=1, keepdims=True)
    var = jnp.mean((u - mu) ** 2, axis=1, keepdims=True)
    t = g_ref[...] * (u - mu) / jnp.sqrt(var + EPS) + be_ref[...]
    o_ref[...] = jnp.maximum(t, 0.0)


_VEC_SPEC = pl.BlockSpec((1, D), lambda i: (0, 0))
_Z_SPEC = pl.BlockSpec((NSC, BN, H), lambda i: (0, i, 0))
_S_SPEC = pl.BlockSpec((BN, 1), lambda i: (i, 0))
_W_SPEC = pl.BlockSpec((D, D), lambda i: (0, 0))

_tc_mm = pl.pallas_call(
    _tc_mm_body,
    grid=(NBI,),
    in_specs=[pl.BlockSpec((BN, D), lambda i: (i, 0)), _W_SPEC],
    out_specs=pl.BlockSpec((BN, D), lambda i: (i, 0)),
    out_shape=jax.ShapeDtypeStruct((NP, D), jnp.float32),
)

_tc_first = pl.pallas_call(
    _tc_first_body,
    grid=(NBI,),
    in_specs=[
        pl.BlockSpec((BN, D), lambda i: (i, 0)),
        pl.BlockSpec((NSC, BN, H), lambda i: (0, i, 0)),
    ],
    out_specs=[_Z_SPEC, _S_SPEC],
    out_shape=[
        jax.ShapeDtypeStruct((NSC, NP, H), jnp.float32),
        jax.ShapeDtypeStruct((NP, 1), jnp.float32),
    ],
)

_tc_mid = pl.pallas_call(
    _tc_mid_body,
    grid=(NBI,),
    in_specs=[_Z_SPEC, _S_SPEC, _VEC_SPEC, _VEC_SPEC, _VEC_SPEC, _W_SPEC],
    out_specs=_Z_SPEC,
    out_shape=jax.ShapeDtypeStruct((NSC, NP, H), jnp.float32),
)

_tc_final = pl.pallas_call(
    _tc_final_body,
    grid=(NBI,),
    in_specs=[_Z_SPEC, _S_SPEC, _VEC_SPEC, _VEC_SPEC, _VEC_SPEC],
    out_specs=pl.BlockSpec((BN, D), lambda i: (i, 0)),
    out_shape=jax.ShapeDtypeStruct((NP, D), jnp.float32),
)


def kernel(x, edge_index, W1, b1, g1, be1, W2, b2, g2, be2,
           W3, b3, g3, be3, W4, b4, g4, be4):
    f32 = jnp.float32
    ei = edge_index.astype(jnp.int32)
    pad_e = EP - N_EDGES
    rowp = jnp.concatenate([ei[0], jnp.zeros((pad_e,), jnp.int32)])
    colp = jnp.concatenate([ei[1], jnp.full((pad_e,), NP, jnp.int32)])
    rowboth = jnp.stack([rowp, rowp + NP]).reshape(NSC, NT, NB_E, K)
    col_e = colp.reshape(NT, NB_E, K)
    col_d = colp.reshape(NSC * NT, NB_D, K)
    xp = jnp.pad(x, ((0, NP - N_NODES), (0, 0)))

    t1 = _tc_mm(xp, W1)
    dparts = _sc_degree(col_d, jnp.zeros((NP, H), f32), jnp.ones((K, H), f32))
    y, s = _tc_first(t1, dparts)

    params = [(b1, g1, be1, W2), (b2, g2, be2, W3), (b3, g3, be3, W4)]
    for (b, g, be, w_next) in params:
        z = _sc_edge_sum(rowboth, col_e, y.reshape(NSC * NP, H))
        y = _tc_mid(z, s, b.reshape(1, D), g.reshape(1, D), be.reshape(1, D),
                    w_next)
    z = _sc_edge_sum(rowboth, col_e, y.reshape(NSC * NP, H))
    h = _tc_final(z, s, b4.reshape(1, D), g4.reshape(1, D), be4.reshape(1, D))
    return h[:N_NODES]


# speedup vs baseline: 1.1903x; 1.0004x over previous
import functools

import jax
import jax.numpy as jnp
from jax import lax
from jax.experimental import pallas as pl
from jax.experimental.pallas import tpu as pltpu
from jax.experimental.pallas import tpu_sc as plsc

N_NODES = 10000
NP = 10240
NPA = NP + 16
N_EDGES = 160000
EP = 163840
D = 256
H = 128
EPS = 1e-5
K = 128
NSC = 2
NT = 16
RPT = NP // NT
BN = 512
NBI = NP // BN

ET_E = EP // NT
NB_E = ET_E // K
CH = 40
ET_D = EP // (NSC * NT)
NB_D = ET_D // K

_MESH = plsc.VectorSubcoreMesh(
    core_axis_name="c", subcore_axis_name="s", num_cores=NSC, num_subcores=NT
)


@functools.partial(
    pl.kernel,
    out_type=jax.ShapeDtypeStruct((NSC, NP, H), jnp.float32),
    mesh=_MESH,
    scratch_types=[
        pltpu.VMEM((NB_D, K), jnp.int32),
        pltpu.VMEM((K, H), jnp.float32),
        pltpu.VMEM_SHARED((NPA, H), jnp.float32),
    ],
)
def _sc_degree(col_hbm, zeros_hbm, ones_hbm, out_hbm, idx_v, ones_v, acc):
    cid = lax.axis_index("c")
    sid = lax.axis_index("s")
    pltpu.sync_copy(zeros_hbm.at[pl.ds(sid * RPT, RPT)],
                    acc.at[pl.ds(sid * RPT, RPT)])
    pltpu.sync_copy(ones_hbm, ones_v)
    tile = cid * NT + sid
    pltpu.sync_copy(col_hbm.at[tile], idx_v)
    plsc.subcore_barrier()

    def body(b, carry):
        pltpu.sync_copy(ones_v, acc.at[idx_v.at[b]], add=True)
        return carry

    lax.fori_loop(0, NB_D, body, 0)
    plsc.subcore_barrier()
    pltpu.sync_copy(acc.at[pl.ds(sid * RPT, RPT)],
                    out_hbm.at[cid, pl.ds(sid * RPT, RPT)])


@functools.partial(
    pl.kernel,
    out_type=jax.ShapeDtypeStruct((NSC, NP, H), jnp.float32),
    mesh=_MESH,
    scratch_types=[
        pltpu.VMEM((CH, K), jnp.int32),
        pltpu.VMEM((CH, K), jnp.int32),
        pltpu.VMEM((K, H), jnp.float32),
        pltpu.VMEM((K, H), jnp.float32),
        pltpu.VMEM_SHARED((NPA, H), jnp.float32),
        pltpu.SemaphoreType.DMA,
        pltpu.SemaphoreType.DMA,
    ],
)
def _sc_edge_sum(row_hbm, col_hbm, y_hbm, out_hbm, idxr, idxc, gbuf0, gbuf1,
                 acc, sem0, sem1):
    cid = lax.axis_index("c")
    sid = lax.axis_index("s")
    pltpu.sync_copy(y_hbm.at[pl.ds(cid * NP + sid * RPT, RPT)],
                    acc.at[pl.ds(sid * RPT, RPT)])
    plsc.subcore_barrier()

    def chunk(ch, carry):
        pltpu.sync_copy(row_hbm.at[cid, sid, pl.ds(ch * CH, CH)], idxr)
        pltpu.sync_copy(col_hbm.at[sid, pl.ds(ch * CH, CH)], idxc)
        pltpu.async_copy(y_hbm.at[idxr.at[0]], gbuf0, sem0)

        def body(i, c2):
            b0 = 2 * i
            pltpu.async_copy(y_hbm.at[idxr.at[b0 + 1]], gbuf1, sem1)
            pltpu.make_async_copy(y_hbm.at[idxr.at[b0]], gbuf0, sem0).wait()
            pltpu.sync_copy(gbuf0, acc.at[idxc.at[b0]], add=True)

            @pl.when(b0 + 2 < CH)
            def _():
                pltpu.async_copy(y_hbm.at[idxr.at[b0 + 2]], gbuf0, sem0)

            pltpu.make_async_copy(y_hbm.at[idxr.at[b0 + 1]], gbuf1, sem1).wait()
            pltpu.sync_copy(gbuf1, acc.at[idxc.at[b0 + 1]], add=True)
            return c2

        lax.fori_loop(0, CH // 2, body, 0)
        return carry

    lax.fori_loop(0, NB_E // CH, chunk, 0)
    plsc.subcore_barrier()
    pltpu.sync_copy(acc.at[pl.ds(sid * RPT, RPT)],
                    out_hbm.at[cid, pl.ds(sid * RPT, RPT)])


def _tc_mm_body(x_ref, w_ref, o_ref):
    o_ref[...] = jnp.dot(x_ref[...], w_ref[...],
                         preferred_element_type=jnp.float32)


def _tc_first_body(t_ref, dp_ref, y_ref, s_ref):
    deg = jnp.sum(dp_ref[...], axis=(0, 2)) * (1.0 / H) + 1.0
    s = (1.0 / jnp.sqrt(deg))[:, None]
    y = t_ref[...] * s
    y_ref[0] = y[:, :H]
    y_ref[1] = y[:, H:]
    s_ref[...] = s


def _tc_mid_body(z_ref, s_ref, b_ref, g_ref, be_ref, w_ref, y_ref):
    s = s_ref[...]
    u = jnp.concatenate([z_ref[0], z_ref[1]], axis=1) * s + b_ref[...]
    mu = jnp.mean(u, axis=1, keepdims=True)
    var = jnp.mean((u - mu) ** 2, axis=1, keepdims=True)
    t = g_ref[...] * (u - mu) / jnp.sqrt(var + EPS) + be_ref[...]
    t = jnp.maximum(t, 0.0)
    y = jnp.dot(t, w_ref[...], preferred_element_type=jnp.float32) * s
    y_ref[0] = y[:, :H]
    y_ref[1] = y[:, H:]


def _tc_final_body(z_ref, s_ref, b_ref, g_ref, be_ref, o_ref):
    s = s_ref[...]
    u = jnp.concatenate([z_ref[0], z_ref[1]], axis=1) * s + b_ref[...]
    mu = jnp.mean(u, axis=1, keepdims=True)
    var = jnp.mean((u - mu) ** 2, axis=1, keepdims=True)
    t = g_ref[...] * (u - mu) / jnp.sqrt(var + EPS) + be_ref[...]
    o_ref[...] = jnp.maximum(t, 0.0)


_VEC_SPEC = pl.BlockSpec((1, D), lambda i: (0, 0))
_Z_SPEC = pl.BlockSpec((NSC, BN, H), lambda i: (0, i, 0))
_S_SPEC = pl.BlockSpec((BN, 1), lambda i: (i, 0))
_W_SPEC = pl.BlockSpec((D, D), lambda i: (0, 0))

_tc_mm = pl.pallas_call(
    _tc_mm_body,
    grid=(NBI,),
    in_specs=[pl.BlockSpec((BN, D), lambda i: (i, 0)), _W_SPEC],
    out_specs=pl.BlockSpec((BN, D), lambda i: (i, 0)),
    out_shape=jax.ShapeDtypeStruct((NP, D), jnp.float32),
)

_tc_first = pl.pallas_call(
    _tc_first_body,
    grid=(NBI,),
    in_specs=[
        pl.BlockSpec((BN, D), lambda i: (i, 0)),
        pl.BlockSpec((NSC, BN, H), lambda i: (0, i, 0)),
    ],
    out_specs=[_Z_SPEC, _S_SPEC],
    out_shape=[
        jax.ShapeDtypeStruct((NSC, NP, H), jnp.float32),
        jax.ShapeDtypeStruct((NP, 1), jnp.float32),
    ],
)

_tc_mid = pl.pallas_call(
    _tc_mid_body,
    grid=(NBI,),
    in_specs=[_Z_SPEC, _S_SPEC, _VEC_SPEC, _VEC_SPEC, _VEC_SPEC, _W_SPEC],
    out_specs=_Z_SPEC,
    out_shape=jax.ShapeDtypeStruct((NSC, NP, H), jnp.float32),
)

_tc_final = pl.pallas_call(
    _tc_final_body,
    grid=(NBI,),
    in_specs=[_Z_SPEC, _S_SPEC, _VEC_SPEC, _VEC_SPEC, _VEC_SPEC],
    out_specs=pl.BlockSpec((BN, D), lambda i: (i, 0)),
    out_shape=jax.ShapeDtypeStruct((NP, D), jnp.float32),
)


def kernel(x, edge_index, W1, b1, g1, be1, W2, b2, g2, be2,
           W3, b3, g3, be3, W4, b4, g4, be4):
    f32 = jnp.float32
    ei = edge_index.astype(jnp.int32)
    pad_e = EP - N_EDGES
    rowp = jnp.concatenate([ei[0], jnp.zeros((pad_e,), jnp.int32)])
    colp = jnp.concatenate([ei[1], jnp.full((pad_e,), NP, jnp.int32)])
    rowboth = jnp.stack([rowp, rowp + NP]).reshape(NSC, NT, NB_E, K)
    col_e = colp.reshape(NT, NB_E, K)
    col_d = colp.reshape(NSC * NT, NB_D, K)
    xp = jnp.pad(x, ((0, NP - N_NODES), (0, 0)))

    t1 = _tc_mm(xp, W1)
    dparts = _sc_degree(col_d, jnp.zeros((NP, H), f32), jnp.ones((K, H), f32))
    y, s = _tc_first(t1, dparts)

    params = [(b1, g1, be1, W2), (b2, g2, be2, W3), (b3, g3, be3, W4)]
    for (b, g, be, w_next) in params:
        z = _sc_edge_sum(rowboth, col_e, y.reshape(NSC * NP, H))
        y = _tc_mid(z, s, b.reshape(1, D), g.reshape(1, D), be.reshape(1, D),
                    w_next)
    z = _sc_edge_sum(rowboth, col_e, y.reshape(NSC * NP, H))
    h = _tc_final(z, s, b4.reshape(1, D), g4.reshape(1, D), be4.reshape(1, D))
    return h[:N_NODES]
